# baseline (speedup 1.0000x reference)
import functools

import jax
import jax.numpy as jnp
from jax import lax
from jax.experimental import pallas as pl
from jax.experimental.pallas import tpu as pltpu
from jax.experimental.pallas import tpu_sc as plsc

N = 10000
D = 128
E = 320000

NC = 2
NS = 16
NW = NC * NS
L = 16

CHUNK = 128
TCH = 80
BLK = 16
GCH = NW * TCH
N1 = 10240
RPT = N1 // NS
PAD = GCH * CHUNK - E

_mesh = plsc.VectorSubcoreMesh(
    core_axis_name="c", subcore_axis_name="s", num_cores=NC, num_subcores=NS
)


@functools.partial(
    pl.kernel,
    out_type=jax.ShapeDtypeStruct((NC, N1), jnp.float32),
    mesh=_mesh,
    scratch_types=[
        pltpu.VMEM((TCH, CHUNK), jnp.int32),
        pltpu.VMEM((128,), jnp.float32),
        pltpu.VMEM((RPT,), jnp.float32),
        pltpu.VMEM_SHARED((N1,), jnp.float32),
    ],
)
def _deg_kernel(ei_hbm, out_hbm, idx_v, ones_v, zero_v, deg_sh):
    c = lax.axis_index("c")
    s = lax.axis_index("s")
    w = c * NS + s

    for i in range(128 // L):
        ones_v[pl.ds(i * L, L)] = jnp.ones((L,), jnp.float32)
    for i in range(RPT // L):
        zero_v[pl.ds(i * L, L)] = jnp.zeros((L,), jnp.float32)

    pltpu.sync_copy(ei_hbm.at[1, pl.ds(w * TCH, TCH)], idx_v)
    pltpu.sync_copy(zero_v, deg_sh.at[pl.ds(s * RPT, RPT)])
    plsc.subcore_barrier()

    def body(j, carry):
        pltpu.sync_copy(ones_v.at[pl.ds(0, CHUNK)], deg_sh.at[idx_v.at[j]],
                        add=True)
        return carry

    lax.fori_loop(0, TCH, body, 0)
    plsc.subcore_barrier()
    pltpu.sync_copy(deg_sh.at[pl.ds(s * RPT, RPT)],
                    out_hbm.at[c, pl.ds(s * RPT, RPT)])


@functools.partial(
    pl.kernel,
    out_type=jax.ShapeDtypeStruct((NC, N1, D), jnp.float32),
    mesh=_mesh,
    scratch_types=[
        pltpu.VMEM((TCH, CHUNK), jnp.int32),
        pltpu.VMEM((BLK, CHUNK), jnp.int32),
        pltpu.VMEM((2, CHUNK, D), jnp.float32),
        pltpu.VMEM_SHARED((N1, D), jnp.float32),
        pltpu.SemaphoreType.DMA,
        pltpu.SemaphoreType.DMA,
        pltpu.SemaphoreType.DMA,
        pltpu.SemaphoreType.DMA,
    ],
)
def _agg_kernel(y_hbm, ei_hbm, out_hbm, src_v, dst_v, buf_v,
                acc_sh, gsem0, gsem1, ssem0, ssem1):
    c = lax.axis_index("c")
    s = lax.axis_index("s")
    w = c * NS + s
    base = s * RPT

    pltpu.sync_copy(ei_hbm.at[0, pl.ds(w * TCH, TCH)], src_v)
    pltpu.sync_copy(y_hbm.at[pl.ds(base, RPT)], acc_sh.at[pl.ds(base, RPT)])
    plsc.subcore_barrier()

    def gather_start(j, b, sem):
        pltpu.async_copy(y_hbm.at[src_v.at[j]], buf_v.at[b], sem)

    def gather_wait(j, b, sem):
        pltpu.make_async_copy(y_hbm.at[src_v.at[j]], buf_v.at[b], sem).wait()

    def scatter_start(jb, b, sem):
        pltpu.async_copy(buf_v.at[b], acc_sh.at[dst_v.at[jb]], sem, add=True)

    def scatter_wait(jb, b, sem):
        pltpu.make_async_copy(buf_v.at[b], acc_sh.at[dst_v.at[jb]],
                              sem).wait()

    gather_start(0, 0, gsem0)
    gather_start(1, 1, gsem1)

    last_blk = TCH // BLK - 1
    for blk in range(TCH // BLK):
        pltpu.sync_copy(ei_hbm.at[1, pl.ds(w * TCH + blk * BLK, BLK)], dst_v)

        def pair(jj, carry, blk=blk):
            j0 = blk * BLK + 2 * jj
            j1 = j0 + 1
            gather_wait(j0, 0, gsem0)
            pltpu.sync_copy(buf_v.at[0], acc_sh.at[dst_v.at[2 * jj]],
                            add=True)
            if blk < last_blk:
                gather_start(j0 + 2, 0, gsem0)
            else:
                @pl.when(jj < BLK // 2 - 1)
                def _():
                    gather_start(j0 + 2, 0, gsem0)
            gather_wait(j1, 1, gsem1)
            pltpu.sync_copy(buf_v.at[1], acc_sh.at[dst_v.at[2 * jj + 1]],
                            add=True)
            if blk < last_blk:
                gather_start(j1 + 2, 1, gsem1)
            else:
                @pl.when(jj < BLK // 2 - 1)
                def _():
                    gather_start(j1 + 2, 1, gsem1)
            return carry

        lax.fori_loop(0, BLK // 2, pair, 0)

    plsc.subcore_barrier()
    pltpu.sync_copy(acc_sh.at[pl.ds(base, RPT)],
                    out_hbm.at[c, pl.ds(base, RPT)])


def _xw_body(x_ref, w_ref, xw_ref):
    xw_ref[...] = jnp.dot(x_ref[...], w_ref[...],
                          preferred_element_type=jnp.float32)


def _scale_body(xw_ref, dp_ref, y_ref):
    deg = dp_ref[0] + dp_ref[1] + 1.0
    y_ref[...] = xw_ref[...] * lax.rsqrt(deg)


def _epilogue_body(a_ref, y_ref, dp_ref, x_ref, b_ref, bw_ref, bb_ref,
                   bm_ref, bv_ref, o_ref):
    deg = dp_ref[0] + dp_ref[1] + 1.0
    dinv = lax.rsqrt(deg)
    agg = (a_ref[0] + a_ref[1] - y_ref[...]) * dinv + b_ref[...]
    inv_std = lax.rsqrt(bv_ref[...] + 1e-5)
    h = (agg - bm_ref[...]) * inv_std * bw_ref[...] + bb_ref[...]
    o_ref[...] = jnp.maximum(h, 0.0) + x_ref[...]


BR = 1024
_GRID = N1 // BR


def kernel(x, edge_index, W, b, bn_weight, bn_bias, bn_mean, bn_var):
    pad = (jnp.arange(PAD, dtype=edge_index.dtype) % (N1 - N)) + N
    ei = jnp.concatenate([edge_index, jnp.broadcast_to(pad, (2, PAD))], axis=1)
    ei = ei.reshape(2, GCH, CHUNK)
    x_pad = jnp.pad(x, ((0, N1 - N), (0, 0)))

    deg_parts = _deg_kernel(ei)
    dp3 = deg_parts.reshape(NC, N1, 1)

    xw = pl.pallas_call(
        _xw_body,
        grid=(_GRID,),
        in_specs=[
            pl.BlockSpec((BR, D), lambda i: (i, 0)),
            pl.BlockSpec((D, D), lambda i: (0, 0)),
        ],
        out_specs=pl.BlockSpec((BR, D), lambda i: (i, 0)),
        out_shape=jax.ShapeDtypeStruct((N1, D), jnp.float32),
    )(x_pad, W)

    y = pl.pallas_call(
        _scale_body,
        grid=(_GRID,),
        in_specs=[
            pl.BlockSpec((BR, D), lambda i: (i, 0)),
            pl.BlockSpec((NC, BR, 1), lambda i: (0, i, 0)),
        ],
        out_specs=pl.BlockSpec((BR, D), lambda i: (i, 0)),
        out_shape=jax.ShapeDtypeStruct((N1, D), jnp.float32),
    )(xw, dp3)

    agg_parts = _agg_kernel(y, ei)

    vec = lambda a: a.reshape(1, D)
    h = pl.pallas_call(
        _epilogue_body,
        grid=(_GRID,),
        in_specs=[
            pl.BlockSpec((NC, BR, D), lambda i: (0, i, 0)),
            pl.BlockSpec((BR, D), lambda i: (i, 0)),
            pl.BlockSpec((NC, BR, 1), lambda i: (0, i, 0)),
            pl.BlockSpec((BR, D), lambda i: (i, 0)),
        ] + [pl.BlockSpec((1, D), lambda i: (0, 0))] * 5,
        out_specs=pl.BlockSpec((BR, D), lambda i: (i, 0)),
        out_shape=jax.ShapeDtypeStruct((N1, D), jnp.float32),
    )(agg_parts, y, dp3, x_pad, vec(b), vec(bn_weight), vec(bn_bias),
      vec(bn_mean), vec(bn_var))

    return h[:N]


# speedup vs baseline: 1.0162x; 1.0162x over previous
import functools

import jax
import jax.numpy as jnp
from jax import lax
from jax.experimental import pallas as pl
from jax.experimental.pallas import tpu as pltpu
from jax.experimental.pallas import tpu_sc as plsc

N = 10000
D = 128
E = 320000

NC = 2
NS = 16
NW = NC * NS
L = 16

CHUNK = 125
TCH = 80
BLK = 16
GCH = NW * TCH
N1 = 10240
RPT = N1 // NS

_mesh = plsc.VectorSubcoreMesh(
    core_axis_name="c", subcore_axis_name="s", num_cores=NC, num_subcores=NS
)


@functools.partial(
    pl.kernel,
    out_type=jax.ShapeDtypeStruct((NC, N1), jnp.float32),
    mesh=_mesh,
    scratch_types=[
        pltpu.VMEM((TCH, CHUNK), jnp.int32),
        pltpu.VMEM((128,), jnp.float32),
        pltpu.VMEM((RPT,), jnp.float32),
        pltpu.VMEM_SHARED((N1,), jnp.float32),
    ],
)
def _deg_kernel(ei_hbm, out_hbm, idx_v, ones_v, zero_v, deg_sh):
    c = lax.axis_index("c")
    s = lax.axis_index("s")
    w = c * NS + s

    for i in range(128 // L):
        ones_v[pl.ds(i * L, L)] = jnp.ones((L,), jnp.float32)
    for i in range(RPT // L):
        zero_v[pl.ds(i * L, L)] = jnp.zeros((L,), jnp.float32)

    pltpu.sync_copy(ei_hbm.at[1, pl.ds(w * TCH, TCH)], idx_v)
    pltpu.sync_copy(zero_v, deg_sh.at[pl.ds(s * RPT, RPT)])
    plsc.subcore_barrier()

    def body(j, carry):
        pltpu.sync_copy(ones_v.at[pl.ds(0, CHUNK)], deg_sh.at[idx_v.at[j]],
                        add=True)
        return carry

    lax.fori_loop(0, TCH, body, 0)
    plsc.subcore_barrier()
    pltpu.sync_copy(deg_sh.at[pl.ds(s * RPT, RPT)],
                    out_hbm.at[c, pl.ds(s * RPT, RPT)])


@functools.partial(
    pl.kernel,
    out_type=jax.ShapeDtypeStruct((NC, N1, D), jnp.float32),
    mesh=_mesh,
    scratch_types=[
        pltpu.VMEM((TCH, CHUNK), jnp.int32),
        pltpu.VMEM((BLK, CHUNK), jnp.int32),
        pltpu.VMEM((2, CHUNK, D), jnp.float32),
        pltpu.VMEM_SHARED((N1, D), jnp.float32),
        pltpu.SemaphoreType.DMA,
        pltpu.SemaphoreType.DMA,
        pltpu.SemaphoreType.DMA,
        pltpu.SemaphoreType.DMA,
    ],
)
def _agg_kernel(y_hbm, ei_hbm, out_hbm, src_v, dst_v, buf_v,
                acc_sh, gsem0, gsem1, ssem0, ssem1):
    c = lax.axis_index("c")
    s = lax.axis_index("s")
    w = c * NS + s
    base = s * RPT

    pltpu.sync_copy(ei_hbm.at[0, pl.ds(w * TCH, TCH)], src_v)
    pltpu.sync_copy(y_hbm.at[pl.ds(base, RPT)], acc_sh.at[pl.ds(base, RPT)])
    plsc.subcore_barrier()

    def gather_start(j, b, sem):
        pltpu.async_copy(y_hbm.at[src_v.at[j]], buf_v.at[b], sem)

    def gather_wait(j, b, sem):
        pltpu.make_async_copy(y_hbm.at[src_v.at[j]], buf_v.at[b], sem).wait()

    def scatter_start(jb, b, sem):
        pltpu.async_copy(buf_v.at[b], acc_sh.at[dst_v.at[jb]], sem, add=True)

    def scatter_wait(jb, b, sem):
        pltpu.make_async_copy(buf_v.at[b], acc_sh.at[dst_v.at[jb]],
                              sem).wait()

    gather_start(0, 0, gsem0)
    gather_start(1, 1, gsem1)

    last_blk = TCH // BLK - 1
    for blk in range(TCH // BLK):
        pltpu.sync_copy(ei_hbm.at[1, pl.ds(w * TCH + blk * BLK, BLK)], dst_v)

        def pair(jj, carry, blk=blk):
            j0 = blk * BLK + 2 * jj
            j1 = j0 + 1
            gather_wait(j0, 0, gsem0)
            pltpu.sync_copy(buf_v.at[0], acc_sh.at[dst_v.at[2 * jj]],
                            add=True)
            if blk < last_blk:
                gather_start(j0 + 2, 0, gsem0)
            else:
                @pl.when(jj < BLK // 2 - 1)
                def _():
                    gather_start(j0 + 2, 0, gsem0)
            gather_wait(j1, 1, gsem1)
            pltpu.sync_copy(buf_v.at[1], acc_sh.at[dst_v.at[2 * jj + 1]],
                            add=True)
            if blk < last_blk:
                gather_start(j1 + 2, 1, gsem1)
            else:
                @pl.when(jj < BLK // 2 - 1)
                def _():
                    gather_start(j1 + 2, 1, gsem1)
            return carry

        lax.fori_loop(0, BLK // 2, pair, 0)

    plsc.subcore_barrier()
    pltpu.sync_copy(acc_sh.at[pl.ds(base, RPT)],
                    out_hbm.at[c, pl.ds(base, RPT)])


def _prescale_body(x_ref, w_ref, dp_ref, y_ref):
    xw = jnp.dot(x_ref[...], w_ref[...], preferred_element_type=jnp.float32)
    deg = dp_ref[0] + dp_ref[1] + 1.0
    y_ref[...] = xw * lax.rsqrt(deg)


def _epilogue_body(a_ref, y_ref, dp_ref, x_ref, b_ref, bw_ref, bb_ref,
                   bm_ref, bv_ref, o_ref):
    deg = dp_ref[0] + dp_ref[1] + 1.0
    dinv = lax.rsqrt(deg)
    agg = (a_ref[0] + a_ref[1] - y_ref[...]) * dinv + b_ref[...]
    inv_std = lax.rsqrt(bv_ref[...] + 1e-5)
    h = (agg - bm_ref[...]) * inv_std * bw_ref[...] + bb_ref[...]
    o_ref[...] = jnp.maximum(h, 0.0) + x_ref[...]


BR = 1024
_GRID = N1 // BR


def kernel(x, edge_index, W, b, bn_weight, bn_bias, bn_mean, bn_var):
    ei = edge_index.reshape(2, GCH, CHUNK)
    x_pad = jnp.pad(x, ((0, N1 - N), (0, 0)))

    deg_parts = _deg_kernel(ei)
    dp3 = deg_parts.reshape(NC, N1, 1)

    y = pl.pallas_call(
        _prescale_body,
        grid=(_GRID,),
        in_specs=[
            pl.BlockSpec((BR, D), lambda i: (i, 0)),
            pl.BlockSpec((D, D), lambda i: (0, 0)),
            pl.BlockSpec((NC, BR, 1), lambda i: (0, i, 0)),
        ],
        out_specs=pl.BlockSpec((BR, D), lambda i: (i, 0)),
        out_shape=jax.ShapeDtypeStruct((N1, D), jnp.float32),
    )(x_pad, W, dp3)

    agg_parts = _agg_kernel(y, ei)

    vec = lambda a: a.reshape(1, D)
    h = pl.pallas_call(
        _epilogue_body,
        grid=(_GRID,),
        in_specs=[
            pl.BlockSpec((NC, BR, D), lambda i: (0, i, 0)),
            pl.BlockSpec((BR, D), lambda i: (i, 0)),
            pl.BlockSpec((NC, BR, 1), lambda i: (0, i, 0)),
            pl.BlockSpec((BR, D), lambda i: (i, 0)),
        ] + [pl.BlockSpec((1, D), lambda i: (0, 0))] * 5,
        out_specs=pl.BlockSpec((BR, D), lambda i: (i, 0)),
        out_shape=jax.ShapeDtypeStruct((N1, D), jnp.float32),
    )(agg_parts, y, dp3, x_pad, vec(b), vec(bn_weight), vec(bn_bias),
      vec(bn_mean), vec(bn_var))

    return h[:N]


# speedup vs baseline: 1.0509x; 1.0342x over previous
import functools

import jax
import jax.numpy as jnp
from jax import lax
from jax.experimental import pallas as pl
from jax.experimental.pallas import tpu as pltpu
from jax.experimental.pallas import tpu_sc as plsc

N = 10000
D = 128
E = 320000

NC = 2
NS = 16
NW = NC * NS
L = 16

CHUNK = 125
TCH = 80
BLK = 16
GCH = NW * TCH
N1 = 10240
RPT = N1 // NS

_mesh = plsc.VectorSubcoreMesh(
    core_axis_name="c", subcore_axis_name="s", num_cores=NC, num_subcores=NS
)


@functools.partial(
    pl.kernel,
    out_type=jax.ShapeDtypeStruct((NC, N1), jnp.float32),
    mesh=_mesh,
    scratch_types=[
        pltpu.VMEM((TCH, CHUNK), jnp.int32),
        pltpu.VMEM((128,), jnp.float32),
        pltpu.VMEM((RPT,), jnp.float32),
        pltpu.VMEM_SHARED((N1,), jnp.float32),
        pltpu.SemaphoreType.DMA,
    ],
)
def _deg_kernel(ei_hbm, out_hbm, idx_v, ones_v, zero_v, deg_sh, dsem):
    c = lax.axis_index("c")
    s = lax.axis_index("s")
    w = c * NS + s

    for i in range(128 // L):
        ones_v[pl.ds(i * L, L)] = jnp.ones((L,), jnp.float32)
    for i in range(RPT // L):
        zero_v[pl.ds(i * L, L)] = jnp.zeros((L,), jnp.float32)

    pltpu.sync_copy(ei_hbm.at[1, pl.ds(w * TCH, TCH)], idx_v)
    pltpu.sync_copy(zero_v, deg_sh.at[pl.ds(s * RPT, RPT)])
    plsc.subcore_barrier()

    def fire(j, carry):
        pltpu.async_copy(ones_v.at[pl.ds(0, CHUNK)], deg_sh.at[idx_v.at[j]],
                         dsem, add=True)
        return carry

    def drain(j, carry):
        pltpu.make_async_copy(ones_v.at[pl.ds(0, CHUNK)],
                              deg_sh.at[idx_v.at[0]], dsem).wait()
        return carry

    lax.fori_loop(0, TCH, fire, 0)
    lax.fori_loop(0, TCH, drain, 0)
    plsc.subcore_barrier()
    pltpu.sync_copy(deg_sh.at[pl.ds(s * RPT, RPT)],
                    out_hbm.at[c, pl.ds(s * RPT, RPT)])


@functools.partial(
    pl.kernel,
    out_type=jax.ShapeDtypeStruct((NC, N1, D), jnp.float32),
    mesh=_mesh,
    scratch_types=[
        pltpu.VMEM((TCH, CHUNK), jnp.int32),
        pltpu.VMEM((BLK, CHUNK), jnp.int32),
        pltpu.VMEM((2, CHUNK, D), jnp.float32),
        pltpu.VMEM_SHARED((N1, D), jnp.float32),
        pltpu.SemaphoreType.DMA,
        pltpu.SemaphoreType.DMA,
        pltpu.SemaphoreType.DMA,
        pltpu.SemaphoreType.DMA,
    ],
)
def _agg_kernel(y_hbm, ei_hbm, out_hbm, src_v, dst_v, buf_v,
                acc_sh, gsem0, gsem1, ssem0, ssem1):
    c = lax.axis_index("c")
    s = lax.axis_index("s")
    w = c * NS + s
    base = s * RPT

    pltpu.sync_copy(ei_hbm.at[0, pl.ds(w * TCH, TCH)], src_v)
    pltpu.sync_copy(y_hbm.at[pl.ds(base, RPT)], acc_sh.at[pl.ds(base, RPT)])
    plsc.subcore_barrier()

    def gather_start(j, b, sem):
        pltpu.async_copy(y_hbm.at[src_v.at[j]], buf_v.at[b], sem)

    def gather_wait(j, b, sem):
        pltpu.make_async_copy(y_hbm.at[src_v.at[j]], buf_v.at[b], sem).wait()

    def scatter_start(jb, b, sem):
        pltpu.async_copy(buf_v.at[b], acc_sh.at[dst_v.at[jb]], sem, add=True)

    def scatter_wait(jb, b, sem):
        pltpu.make_async_copy(buf_v.at[b], acc_sh.at[dst_v.at[jb]],
                              sem).wait()

    gather_start(0, 0, gsem0)
    gather_start(1, 1, gsem1)

    last_blk = TCH // BLK - 1
    for blk in range(TCH // BLK):
        pltpu.sync_copy(ei_hbm.at[1, pl.ds(w * TCH + blk * BLK, BLK)], dst_v)

        def pair(jj, carry, blk=blk):
            j0 = blk * BLK + 2 * jj
            j1 = j0 + 1
            gather_wait(j0, 0, gsem0)
            pltpu.sync_copy(buf_v.at[0], acc_sh.at[dst_v.at[2 * jj]],
                            add=True)
            if blk < last_blk:
                gather_start(j0 + 2, 0, gsem0)
            else:
                @pl.when(jj < BLK // 2 - 1)
                def _():
                    gather_start(j0 + 2, 0, gsem0)
            gather_wait(j1, 1, gsem1)
            pltpu.sync_copy(buf_v.at[1], acc_sh.at[dst_v.at[2 * jj + 1]],
                            add=True)
            if blk < last_blk:
                gather_start(j1 + 2, 1, gsem1)
            else:
                @pl.when(jj < BLK // 2 - 1)
                def _():
                    gather_start(j1 + 2, 1, gsem1)
            return carry

        lax.fori_loop(0, BLK // 2, pair, 0)

    plsc.subcore_barrier()
    pltpu.sync_copy(acc_sh.at[pl.ds(base, RPT)],
                    out_hbm.at[c, pl.ds(base, RPT)])


def _prescale_body(x_ref, w_ref, dp_ref, y_ref):
    xw = jnp.dot(x_ref[...], w_ref[...], preferred_element_type=jnp.float32)
    deg = dp_ref[0] + dp_ref[1] + 1.0
    y_ref[...] = xw * lax.rsqrt(deg)


def _epilogue_body(a_ref, y_ref, dp_ref, x_ref, b_ref, bw_ref, bb_ref,
                   bm_ref, bv_ref, o_ref):
    deg = dp_ref[0] + dp_ref[1] + 1.0
    dinv = lax.rsqrt(deg)
    agg = (a_ref[0] + a_ref[1] - y_ref[...]) * dinv + b_ref[...]
    inv_std = lax.rsqrt(bv_ref[...] + 1e-5)
    h = (agg - bm_ref[...]) * inv_std * bw_ref[...] + bb_ref[...]
    o_ref[...] = jnp.maximum(h, 0.0) + x_ref[...]


BR = 1024
_GRID = N1 // BR


def kernel(x, edge_index, W, b, bn_weight, bn_bias, bn_mean, bn_var):
    ei = edge_index.reshape(2, GCH, CHUNK)

    deg_parts = _deg_kernel(ei)
    dp3 = deg_parts.reshape(NC, N1, 1)

    y = pl.pallas_call(
        _prescale_body,
        grid=(_GRID,),
        in_specs=[
            pl.BlockSpec((BR, D), lambda i: (i, 0)),
            pl.BlockSpec((D, D), lambda i: (0, 0)),
            pl.BlockSpec((NC, BR, 1), lambda i: (0, i, 0)),
        ],
        out_specs=pl.BlockSpec((BR, D), lambda i: (i, 0)),
        out_shape=jax.ShapeDtypeStruct((N1, D), jnp.float32),
    )(x, W, dp3)

    agg_parts = _agg_kernel(y, ei)

    vec = lambda a: a.reshape(1, D)
    h = pl.pallas_call(
        _epilogue_body,
        grid=(_GRID,),
        in_specs=[
            pl.BlockSpec((NC, BR, D), lambda i: (0, i, 0)),
            pl.BlockSpec((BR, D), lambda i: (i, 0)),
            pl.BlockSpec((NC, BR, 1), lambda i: (0, i, 0)),
            pl.BlockSpec((BR, D), lambda i: (i, 0)),
        ] + [pl.BlockSpec((1, D), lambda i: (0, 0))] * 5,
        out_specs=pl.BlockSpec((BR, D), lambda i: (i, 0)),
        out_shape=jax.ShapeDtypeStruct((N1, D), jnp.float32),
    )(agg_parts, y, dp3, x, vec(b), vec(bn_weight), vec(bn_bias),
      vec(bn_mean), vec(bn_var))

    return h[:N]


# speedup vs baseline: 1.0679x; 1.0162x over previous
import functools

import jax
import jax.numpy as jnp
from jax import lax
from jax.experimental import pallas as pl
from jax.experimental.pallas import tpu as pltpu
from jax.experimental.pallas import tpu_sc as plsc

N = 10000
D = 128
E = 320000

NC = 2
NS = 16
NW = NC * NS
L = 16

CHUNK = 125
TCH = 80
BLK = 16
GCH = NW * TCH
N1 = 10240
RPT = N1 // NS

_mesh = plsc.VectorSubcoreMesh(
    core_axis_name="c", subcore_axis_name="s", num_cores=NC, num_subcores=NS
)


@functools.partial(
    pl.kernel,
    out_type=jax.ShapeDtypeStruct((NC, N1), jnp.float32),
    mesh=_mesh,
    scratch_types=[
        pltpu.VMEM((TCH, CHUNK), jnp.int32),
        pltpu.VMEM((128,), jnp.float32),
        pltpu.VMEM((RPT,), jnp.float32),
        pltpu.VMEM_SHARED((N1,), jnp.float32),
        pltpu.SemaphoreType.DMA,
    ],
)
def _deg_kernel(ei_hbm, out_hbm, idx_v, ones_v, zero_v, deg_sh, dsem):
    c = lax.axis_index("c")
    s = lax.axis_index("s")
    w = c * NS + s

    for i in range(128 // L):
        ones_v[pl.ds(i * L, L)] = jnp.ones((L,), jnp.float32)
    for i in range(RPT // L):
        zero_v[pl.ds(i * L, L)] = jnp.zeros((L,), jnp.float32)

    pltpu.sync_copy(ei_hbm.at[1, pl.ds(w * TCH, TCH)], idx_v)
    pltpu.sync_copy(zero_v, deg_sh.at[pl.ds(s * RPT, RPT)])
    plsc.subcore_barrier()

    def fire(j, carry):
        pltpu.async_copy(ones_v.at[pl.ds(0, CHUNK)], deg_sh.at[idx_v.at[j]],
                         dsem, add=True)
        return carry

    def drain(j, carry):
        pltpu.make_async_copy(ones_v.at[pl.ds(0, CHUNK)],
                              deg_sh.at[idx_v.at[0]], dsem).wait()
        return carry

    lax.fori_loop(0, TCH, fire, 0)
    lax.fori_loop(0, TCH, drain, 0)
    plsc.subcore_barrier()
    pltpu.sync_copy(deg_sh.at[pl.ds(s * RPT, RPT)],
                    out_hbm.at[c, pl.ds(s * RPT, RPT)])


@functools.partial(
    pl.kernel,
    out_type=jax.ShapeDtypeStruct((NC, N1, D), jnp.float32),
    mesh=_mesh,
    scratch_types=[
        pltpu.VMEM((TCH, CHUNK), jnp.int32),
        pltpu.VMEM((BLK, CHUNK), jnp.int32),
        pltpu.VMEM((2, CHUNK, D), jnp.float32),
        pltpu.VMEM_SHARED((N1, D), jnp.float32),
        pltpu.SemaphoreType.DMA,
        pltpu.SemaphoreType.DMA,
        pltpu.SemaphoreType.DMA,
        pltpu.SemaphoreType.DMA,
    ],
)
def _agg_kernel(y_hbm, ei_hbm, out_hbm, src_v, dst_v, buf_v,
                acc_sh, gsem0, gsem1, ssem0, ssem1):
    c = lax.axis_index("c")
    s = lax.axis_index("s")
    w = c * NS + s
    base = s * RPT

    pltpu.sync_copy(ei_hbm.at[0, pl.ds(w * TCH, TCH)], src_v)
    pltpu.sync_copy(y_hbm.at[pl.ds(base, RPT)], acc_sh.at[pl.ds(base, RPT)])
    plsc.subcore_barrier()

    def gather_start(j, b, sem):
        pltpu.async_copy(y_hbm.at[src_v.at[j]], buf_v.at[b], sem)

    def gather_wait(j, b, sem):
        pltpu.make_async_copy(y_hbm.at[src_v.at[j]], buf_v.at[b], sem).wait()

    def scatter_start(jb, b, sem):
        pltpu.async_copy(buf_v.at[b], acc_sh.at[dst_v.at[jb]], sem, add=True)

    def scatter_wait(jb, b, sem):
        pltpu.make_async_copy(buf_v.at[b], acc_sh.at[dst_v.at[jb]],
                              sem).wait()

    gather_start(0, 0, gsem0)
    gather_start(1, 1, gsem1)

    last_blk = TCH // BLK - 1
    for blk in range(TCH // BLK):
        pltpu.sync_copy(ei_hbm.at[1, pl.ds(w * TCH + blk * BLK, BLK)], dst_v)

        def pair(jj, carry, blk=blk):
            j0 = blk * BLK + 2 * jj
            j1 = j0 + 1
            gather_wait(j0, 0, gsem0)
            pltpu.sync_copy(buf_v.at[0], acc_sh.at[dst_v.at[2 * jj]],
                            add=True)
            if blk < last_blk:
                gather_start(j0 + 2, 0, gsem0)
            else:
                @pl.when(jj < BLK // 2 - 1)
                def _():
                    gather_start(j0 + 2, 0, gsem0)
            gather_wait(j1, 1, gsem1)
            pltpu.sync_copy(buf_v.at[1], acc_sh.at[dst_v.at[2 * jj + 1]],
                            add=True)
            if blk < last_blk:
                gather_start(j1 + 2, 1, gsem1)
            else:
                @pl.when(jj < BLK // 2 - 1)
                def _():
                    gather_start(j1 + 2, 1, gsem1)
            return carry

        lax.fori_loop(0, BLK // 2, pair, 0)

    plsc.subcore_barrier()
    pltpu.sync_copy(acc_sh.at[pl.ds(base, RPT)],
                    out_hbm.at[c, pl.ds(base, RPT)])


def _prescale_body(x_ref, w_ref, dp_ref, y_ref):
    xw = jnp.dot(x_ref[...], w_ref[...], preferred_element_type=jnp.float32)
    deg = dp_ref[0] + dp_ref[1] + 1.0
    y_ref[...] = xw * lax.rsqrt(deg)


def _epilogue_body(a_ref, y_ref, dp_ref, x_ref, b_ref, bw_ref, bb_ref,
                   bm_ref, bv_ref, o_ref):
    deg = dp_ref[0] + dp_ref[1] + 1.0
    dinv = lax.rsqrt(deg)
    agg = (a_ref[0] + a_ref[1] - y_ref[...]) * dinv + b_ref[...]
    inv_std = lax.rsqrt(bv_ref[...] + 1e-5)
    h = (agg - bm_ref[...]) * inv_std * bw_ref[...] + bb_ref[...]
    o_ref[...] = jnp.maximum(h, 0.0) + x_ref[...]


BR = 2048
_GRID = N1 // BR


def kernel(x, edge_index, W, b, bn_weight, bn_bias, bn_mean, bn_var):
    ei = edge_index.reshape(2, GCH, CHUNK)

    deg_parts = _deg_kernel(ei)
    dp3 = deg_parts.reshape(NC, N1, 1)

    y = pl.pallas_call(
        _prescale_body,
        grid=(_GRID,),
        in_specs=[
            pl.BlockSpec((BR, D), lambda i: (i, 0)),
            pl.BlockSpec((D, D), lambda i: (0, 0)),
            pl.BlockSpec((NC, BR, 1), lambda i: (0, i, 0)),
        ],
        out_specs=pl.BlockSpec((BR, D), lambda i: (i, 0)),
        out_shape=jax.ShapeDtypeStruct((N1, D), jnp.float32),
    )(x, W, dp3)

    agg_parts = _agg_kernel(y, ei)

    vec = lambda a: a.reshape(1, D)
    h = pl.pallas_call(
        _epilogue_body,
        grid=(_GRID,),
        in_specs=[
            pl.BlockSpec((NC, BR, D), lambda i: (0, i, 0)),
            pl.BlockSpec((BR, D), lambda i: (i, 0)),
            pl.BlockSpec((NC, BR, 1), lambda i: (0, i, 0)),
            pl.BlockSpec((BR, D), lambda i: (i, 0)),
        ] + [pl.BlockSpec((1, D), lambda i: (0, 0))] * 5,
        out_specs=pl.BlockSpec((BR, D), lambda i: (i, 0)),
        out_shape=jax.ShapeDtypeStruct((N1, D), jnp.float32),
    )(agg_parts, y, dp3, x, vec(b), vec(bn_weight), vec(bn_bias),
      vec(bn_mean), vec(bn_var))

    return h[:N]


# speedup vs baseline: 1.0780x; 1.0095x over previous
import functools

import jax
import jax.numpy as jnp
from jax import lax
from jax.experimental import pallas as pl
from jax.experimental.pallas import tpu as pltpu
from jax.experimental.pallas import tpu_sc as plsc

N = 10000
D = 128
E = 320000

NC = 2
NS = 16
NW = NC * NS
L = 16

CHUNK = 125
TCH = 80
BLK = 16
GCH = NW * TCH
N1 = 10240
RPT = N1 // NS

_mesh = plsc.VectorSubcoreMesh(
    core_axis_name="c", subcore_axis_name="s", num_cores=NC, num_subcores=NS
)


@functools.partial(
    pl.kernel,
    out_type=jax.ShapeDtypeStruct((NC, N1), jnp.float32),
    mesh=_mesh,
    scratch_types=[
        pltpu.VMEM((TCH, CHUNK), jnp.int32),
        pltpu.VMEM((128,), jnp.float32),
        pltpu.VMEM((RPT,), jnp.float32),
        pltpu.VMEM_SHARED((N1,), jnp.float32),
        pltpu.SemaphoreType.DMA,
    ],
)
def _deg_kernel(ei_hbm, out_hbm, idx_v, ones_v, zero_v, deg_sh, dsem):
    c = lax.axis_index("c")
    s = lax.axis_index("s")
    w = c * NS + s

    for i in range(128 // L):
        ones_v[pl.ds(i * L, L)] = jnp.ones((L,), jnp.float32)
    for i in range(RPT // L):
        zero_v[pl.ds(i * L, L)] = jnp.zeros((L,), jnp.float32)

    pltpu.sync_copy(ei_hbm.at[1, pl.ds(w * TCH, TCH)], idx_v)
    pltpu.sync_copy(zero_v, deg_sh.at[pl.ds(s * RPT, RPT)])
    plsc.subcore_barrier()

    def fire(j, carry):
        pltpu.async_copy(ones_v.at[pl.ds(0, CHUNK)], deg_sh.at[idx_v.at[j]],
                         dsem, add=True)
        return carry

    def drain(j, carry):
        pltpu.make_async_copy(ones_v.at[pl.ds(0, CHUNK)],
                              deg_sh.at[idx_v.at[0]], dsem).wait()
        return carry

    lax.fori_loop(0, TCH, fire, 0)
    lax.fori_loop(0, TCH, drain, 0)
    plsc.subcore_barrier()
    pltpu.sync_copy(deg_sh.at[pl.ds(s * RPT, RPT)],
                    out_hbm.at[c, pl.ds(s * RPT, RPT)])


@functools.partial(
    pl.kernel,
    out_type=jax.ShapeDtypeStruct((NC, N1, D), jnp.float32),
    mesh=_mesh,
    scratch_types=[
        pltpu.VMEM((TCH, CHUNK), jnp.int32),
        pltpu.VMEM((BLK, CHUNK), jnp.int32),
        pltpu.VMEM((2, CHUNK, D), jnp.float32),
        pltpu.VMEM_SHARED((N1, D), jnp.float32),
        pltpu.SemaphoreType.DMA,
        pltpu.SemaphoreType.DMA,
        pltpu.SemaphoreType.DMA,
        pltpu.SemaphoreType.DMA,
    ],
)
def _agg_kernel(y_hbm, ei_hbm, out_hbm, src_v, dst_v, buf_v,
                acc_sh, gsem0, gsem1, ssem0, ssem1):
    c = lax.axis_index("c")
    s = lax.axis_index("s")
    w = c * NS + s
    base = s * RPT

    pltpu.sync_copy(ei_hbm.at[0, pl.ds(w * TCH, TCH)], src_v)
    pltpu.sync_copy(y_hbm.at[pl.ds(base, RPT)], acc_sh.at[pl.ds(base, RPT)])
    plsc.subcore_barrier()

    def gather_start(j, b, sem):
        pltpu.async_copy(y_hbm.at[src_v.at[j]], buf_v.at[b], sem)

    def gather_wait(j, b, sem):
        pltpu.make_async_copy(y_hbm.at[src_v.at[j]], buf_v.at[b], sem).wait()

    def scatter_start(jb, b, sem):
        pltpu.async_copy(buf_v.at[b], acc_sh.at[dst_v.at[jb]], sem, add=True)

    def scatter_wait(jb, b, sem):
        pltpu.make_async_copy(buf_v.at[b], acc_sh.at[dst_v.at[jb]],
                              sem).wait()

    gather_start(0, 0, gsem0)
    gather_start(1, 1, gsem1)

    last_blk = TCH // BLK - 1
    for blk in range(TCH // BLK):
        pltpu.sync_copy(ei_hbm.at[1, pl.ds(w * TCH + blk * BLK, BLK)], dst_v)

        def pair(jj, carry, blk=blk):
            j0 = blk * BLK + 2 * jj
            j1 = j0 + 1
            gather_wait(j0, 0, gsem0)
            pltpu.sync_copy(buf_v.at[0], acc_sh.at[dst_v.at[2 * jj]],
                            add=True)
            if blk < last_blk:
                gather_start(j0 + 2, 0, gsem0)
            else:
                @pl.when(jj < BLK // 2 - 1)
                def _():
                    gather_start(j0 + 2, 0, gsem0)
            gather_wait(j1, 1, gsem1)
            pltpu.sync_copy(buf_v.at[1], acc_sh.at[dst_v.at[2 * jj + 1]],
                            add=True)
            if blk < last_blk:
                gather_start(j1 + 2, 1, gsem1)
            else:
                @pl.when(jj < BLK // 2 - 1)
                def _():
                    gather_start(j1 + 2, 1, gsem1)
            return carry

        lax.fori_loop(0, BLK // 2, pair, 0)

    plsc.subcore_barrier()
    pltpu.sync_copy(acc_sh.at[pl.ds(base, RPT)],
                    out_hbm.at[c, pl.ds(base, RPT)])


def _prescale_body(x_ref, w_ref, dp_ref, y_ref):
    xw = jnp.dot(x_ref[...], w_ref[...], preferred_element_type=jnp.float32)
    deg = dp_ref[0] + dp_ref[1] + 1.0
    y_ref[...] = xw * lax.rsqrt(deg)


def _epilogue_body(a_ref, y_ref, dp_ref, x_ref, b_ref, bw_ref, bb_ref,
                   bm_ref, bv_ref, o_ref):
    deg = dp_ref[0] + dp_ref[1] + 1.0
    dinv = lax.rsqrt(deg)
    agg = (a_ref[0] + a_ref[1] - y_ref[...]) * dinv + b_ref[...]
    inv_std = lax.rsqrt(bv_ref[...] + 1e-5)
    h = (agg - bm_ref[...]) * inv_std * bw_ref[...] + bb_ref[...]
    o_ref[...] = jnp.maximum(h, 0.0) + x_ref[...]


BR = 5120
_GRID = N1 // BR


def kernel(x, edge_index, W, b, bn_weight, bn_bias, bn_mean, bn_var):
    ei = edge_index.reshape(2, GCH, CHUNK)

    deg_parts = _deg_kernel(ei)
    dp3 = deg_parts.reshape(NC, N1, 1)

    y = pl.pallas_call(
        _prescale_body,
        grid=(_GRID,),
        in_specs=[
            pl.BlockSpec((BR, D), lambda i: (i, 0)),
            pl.BlockSpec((D, D), lambda i: (0, 0)),
            pl.BlockSpec((NC, BR, 1), lambda i: (0, i, 0)),
        ],
        out_specs=pl.BlockSpec((BR, D), lambda i: (i, 0)),
        out_shape=jax.ShapeDtypeStruct((N1, D), jnp.float32),
    )(x, W, dp3)

    agg_parts = _agg_kernel(y, ei)

    vec = lambda a: a.reshape(1, D)
    h = pl.pallas_call(
        _epilogue_body,
        grid=(_GRID,),
        in_specs=[
            pl.BlockSpec((NC, BR, D), lambda i: (0, i, 0)),
            pl.BlockSpec((BR, D), lambda i: (i, 0)),
            pl.BlockSpec((NC, BR, 1), lambda i: (0, i, 0)),
            pl.BlockSpec((BR, D), lambda i: (i, 0)),
        ] + [pl.BlockSpec((1, D), lambda i: (0, 0))] * 5,
        out_specs=pl.BlockSpec((BR, D), lambda i: (i, 0)),
        out_shape=jax.ShapeDtypeStruct((N1, D), jnp.float32),
    )(agg_parts, y, dp3, x, vec(b), vec(bn_weight), vec(bn_bias),
      vec(bn_mean), vec(bn_var))

    return h[:N]
